# baseline (speedup 1.0000x reference)
import math

import jax
import jax.numpy as jnp
from jax import lax
from jax.experimental import pallas as pl
from jax.experimental.pallas import tpu as pltpu
from jax.experimental.pallas import tpu_sc as plsc

_B, _T = 32, 32768
_N = _B * _T
_NC, _NS = 2, 16
_NW = _NC * _NS
_PER = _N // _NW
_V = 16
_C = 4096
_NCH = _PER // _C

_MAGIC = 12582912.0
_A = math.log(2.0) / 12.0
_BIAS = math.log(440.0) - 69.0 * _A


def _detune_body(p_hbm, d_hbm, w_hbm, out_hbm, p_v, d_v, o_v, w_v,
                 sem_in, sem_out, sem_w):
    wid = lax.axis_index("s") * _NC + lax.axis_index("c")
    base = wid * _PER
    w_desc = pltpu.async_copy(w_hbm, w_v, sem_w)

    def start_in(k):
        sl_h = pl.ds(base + k * _C, _C)
        sl_v = pl.ds(k * _C, _C)
        return (pltpu.async_copy(p_hbm.at[sl_h], p_v.at[sl_v], sem_in.at[k, 0]),
                pltpu.async_copy(d_hbm.at[sl_h], d_v.at[sl_v], sem_in.at[k, 1]))

    in_descs = [start_in(k) for k in range(_NCH)]
    w_desc.wait()
    out_descs = []
    for k in range(_NCH):
        dp, dd = in_descs[k]
        dp.wait()
        dd.wait()

        @plsc.parallel_loop(k * _C, (k + 1) * _C, step=_V, unroll=8)
        def _step(i):
            sl = pl.ds(i, _V)
            p = p_v[sl]
            d = d_v[sl]
            r = (p + _MAGIC) - _MAGIC
            r = jnp.minimum(jnp.maximum(r, 0.0), 127.0)
            idx = r.astype(jnp.int32)
            t = plsc.load_gather(w_v, [idx])
            o_v[sl] = jnp.exp((p + d + t) * _A + _BIAS)

        out_descs.append(pltpu.async_copy(
            o_v.at[pl.ds(k * _C, _C)],
            out_hbm.at[pl.ds(base + k * _C, _C)],
            sem_out.at[k]))
    for dsc in out_descs:
        dsc.wait()


def kernel(extended_pitch, global_detuning, embedding_weight):
    p = extended_pitch.reshape(_N)
    d = global_detuning.reshape(_N)
    w = embedding_weight.reshape(128)
    mesh = plsc.VectorSubcoreMesh(core_axis_name="c", subcore_axis_name="s")
    f = pl.kernel(
        _detune_body,
        out_type=jax.ShapeDtypeStruct((_N,), jnp.float32),
        mesh=mesh,
        compiler_params=pltpu.CompilerParams(needs_layout_passes=False),
        scratch_types=[
            pltpu.VMEM((_PER,), jnp.float32),
            pltpu.VMEM((_PER,), jnp.float32),
            pltpu.VMEM((_PER,), jnp.float32),
            pltpu.VMEM((128,), jnp.float32),
            pltpu.SemaphoreType.DMA((_NCH, 2)),
            pltpu.SemaphoreType.DMA((_NCH,)),
            pltpu.SemaphoreType.DMA,
        ],
    )
    out = f(p, d, w)
    return out.reshape(_B, _T, 1)


# speedup vs baseline: 1.0727x; 1.0727x over previous
import math

import jax
import jax.numpy as jnp
from jax import lax
from jax.experimental import pallas as pl
from jax.experimental.pallas import tpu as pltpu
from jax.experimental.pallas import tpu_sc as plsc

_B, _T = 32, 32768
_N = _B * _T
_NC, _NS = 2, 16
_NW = _NC * _NS
_PER = _N // _NW
_V = 16
_C = 8192
_NCH = _PER // _C

_MAGIC = 12582912.0
_A = math.log(2.0) / 12.0
_BIAS = math.log(440.0) - 69.0 * _A


def _detune_body(p_hbm, d_hbm, w_hbm, out_hbm, p_v, d_v, o_v, w_v,
                 sem_in, sem_out, sem_w):
    wid = lax.axis_index("s") * _NC + lax.axis_index("c")
    base = wid * _PER
    w_desc = pltpu.async_copy(w_hbm, w_v, sem_w)

    def start_in(k):
        sl_h = pl.ds(base + k * _C, _C)
        sl_v = pl.ds(k * _C, _C)
        return (pltpu.async_copy(p_hbm.at[sl_h], p_v.at[sl_v], sem_in.at[k, 0]),
                pltpu.async_copy(d_hbm.at[sl_h], d_v.at[sl_v], sem_in.at[k, 1]))

    in_descs = [start_in(k) for k in range(_NCH)]
    w_desc.wait()
    out_descs = []
    for k in range(_NCH):
        dp, dd = in_descs[k]
        dp.wait()
        dd.wait()

        @plsc.parallel_loop(k * _C, (k + 1) * _C, step=_V, unroll=8)
        def _step(i):
            sl = pl.ds(i, _V)
            p = p_v[sl]
            d = d_v[sl]
            pc = jnp.minimum(jnp.maximum(p, 0.0), 127.0)
            idx = plsc.bitcast(pc + _MAGIC, jnp.int32) & 0x3FFFFF
            t = plsc.load_gather(w_v, [idx])
            o_v[sl] = jnp.exp((p + d + t) * _A + _BIAS)

        out_descs.append(pltpu.async_copy(
            o_v.at[pl.ds(k * _C, _C)],
            out_hbm.at[pl.ds(base + k * _C, _C)],
            sem_out.at[k]))
    for dsc in out_descs:
        dsc.wait()


def kernel(extended_pitch, global_detuning, embedding_weight):
    p = extended_pitch.reshape(_N)
    d = global_detuning.reshape(_N)
    w = embedding_weight.reshape(128)
    mesh = plsc.VectorSubcoreMesh(core_axis_name="c", subcore_axis_name="s")
    f = pl.kernel(
        _detune_body,
        out_type=jax.ShapeDtypeStruct((_N,), jnp.float32),
        mesh=mesh,
        compiler_params=pltpu.CompilerParams(needs_layout_passes=False),
        scratch_types=[
            pltpu.VMEM((_PER,), jnp.float32),
            pltpu.VMEM((_PER,), jnp.float32),
            pltpu.VMEM((_PER,), jnp.float32),
            pltpu.VMEM((128,), jnp.float32),
            pltpu.SemaphoreType.DMA((_NCH, 2)),
            pltpu.SemaphoreType.DMA((_NCH,)),
            pltpu.SemaphoreType.DMA,
        ],
    )
    out = f(p, d, w)
    return out.reshape(_B, _T, 1)
